# Initial kernel scaffold; baseline (speedup 1.0000x reference)
import functools

import jax
import jax.numpy as jnp
from jax import lax
from jax.experimental import pallas as pl
from jax.experimental.pallas import tpu as pltpu
from jax.experimental.pallas import tpu_sc as plsc


def _sc_info():
    try:
        info = plsc.get_sparse_core_info()
        return info.num_cores, info.num_subcores
    except Exception:
        return 2, 16


@functools.partial(jax.jit, static_argnames=("nw", "steps", "sents_per_step", "seq", "d"))
def _gather_pool(idx3, table, *, nw, steps, sents_per_step, seq, d):
    ipg = sents_per_step * seq
    total_sents = nw * steps * sents_per_step
    mesh = plsc.VectorSubcoreMesh(core_axis_name="c", subcore_axis_name="s")

    @functools.partial(
        pl.kernel,
        out_type=jax.ShapeDtypeStruct((total_sents, d), jnp.float32),
        mesh=mesh,
        scratch_types=[
            pltpu.VMEM((ipg,), jnp.int32),
            pltpu.VMEM((ipg, d), jnp.float32),
            pltpu.VMEM((sents_per_step, d), jnp.float32),
            pltpu.SemaphoreType.DMA,
        ],
    )
    def k(idx_hbm, table_hbm, out_hbm, idx_v, rows_v, pool_v, sem):
        c = lax.axis_index("c")
        s = lax.axis_index("s")
        w = s * 2 + c
        base = w * (steps * sents_per_step)

        def body(j, carry):
            pltpu.sync_copy(idx_hbm.at[w, j], idx_v)
            pltpu.async_copy(table_hbm.at[idx_v], rows_v, sem).wait()
            for snt in range(sents_per_step):
                for kk in range(d // 16):
                    col = pl.ds(16 * kk, 16)
                    acc = rows_v[seq * snt, col]
                    for r in range(1, seq):
                        acc = acc + rows_v[seq * snt + r, col]
                    pool_v[snt, col] = acc
            pltpu.sync_copy(
                pool_v, out_hbm.at[pl.ds(base + j * sents_per_step, sents_per_step)]
            )
            return carry

        lax.fori_loop(0, steps, body, 0)

    return k(idx3, table)


@functools.partial(jax.jit, static_argnames=("seq",))
def _project_cosine(pooled1, pooled2, Wt, b2, *, seq):
    B, d = pooled1.shape
    p = Wt.shape[1]
    blk = 1024
    inv = 1.0 / float(seq)

    def body(s1_ref, s2_ref, wt_ref, b_ref, out_ref):
        wt = wt_ref[...]
        bb = b_ref[...]
        s1 = s1_ref[...] * inv
        s2 = s2_ref[...] * inv
        p1 = jnp.dot(s1, wt, preferred_element_type=jnp.float32) + bb
        p2 = jnp.dot(s2, wt, preferred_element_type=jnp.float32) + bb
        d12 = jnp.sum(p1 * p2, axis=1)
        n1 = jnp.maximum(jnp.sqrt(jnp.sum(p1 * p1, axis=1)), 1e-8)
        n2 = jnp.maximum(jnp.sqrt(jnp.sum(p2 * p2, axis=1)), 1e-8)
        out_ref[...] = (d12 / (n1 * n2)).reshape(1, blk)

    out = pl.pallas_call(
        body,
        grid=(B // blk,),
        in_specs=[
            pl.BlockSpec((blk, d), lambda i: (i, 0)),
            pl.BlockSpec((blk, d), lambda i: (i, 0)),
            pl.BlockSpec((d, p), lambda i: (0, 0)),
            pl.BlockSpec((1, p), lambda i: (0, 0)),
        ],
        out_specs=pl.BlockSpec((1, blk), lambda i: (i, 0)),
        out_shape=jax.ShapeDtypeStruct((B // blk, blk), jnp.float32),
    )(pooled1, pooled2, Wt, b2)
    return out.reshape(B)


def kernel(x, table, W, b):
    B, two, seq = x.shape
    assert two == 2
    d = table.shape[1]
    nc, ns = _sc_info()
    nw = nc * ns

    sents = B * 2
    sents_per_step = 2
    steps = sents // (nw * sents_per_step)
    assert steps * nw * sents_per_step == sents

    idx = x.transpose(1, 0, 2).astype(jnp.int32)
    idx3 = idx.reshape(nw, steps, sents_per_step * seq)

    pooled = _gather_pool(
        idx3, table, nw=nw, steps=steps, sents_per_step=sents_per_step, seq=seq, d=d
    )
    pooled1 = pooled[:B]
    pooled2 = pooled[B:]

    Wt = W.T
    b2 = b.reshape(1, -1)
    return _project_cosine(pooled1, pooled2, Wt, b2, seq=seq)


# speedup vs baseline: 2.3436x; 2.3436x over previous
import functools

import jax
import jax.numpy as jnp
from jax import lax
from jax.experimental import pallas as pl
from jax.experimental.pallas import tpu as pltpu
from jax.experimental.pallas import tpu_sc as plsc


def _sc_info():
    try:
        info = plsc.get_sparse_core_info()
        return info.num_cores, info.num_subcores
    except Exception:
        return 2, 16


@functools.partial(jax.jit, static_argnames=("nw", "steps", "sents_per_step", "seq", "d"))
def _gather_pool(idx3, table, *, nw, steps, sents_per_step, seq, d):
    ipg = sents_per_step * seq
    total_sents = nw * steps * sents_per_step
    mesh = plsc.VectorSubcoreMesh(core_axis_name="c", subcore_axis_name="s")

    @functools.partial(
        pl.kernel,
        out_type=jax.ShapeDtypeStruct((total_sents, d), jnp.float32),
        mesh=mesh,
        compiler_params=pltpu.CompilerParams(use_tc_tiling_on_sc=False),
        scratch_types=[
            pltpu.VMEM((ipg,), jnp.int32),
            pltpu.VMEM((ipg, d), jnp.float32),
            pltpu.VMEM((sents_per_step, d), jnp.float32),
            pltpu.SemaphoreType.DMA,
        ],
    )
    def k(idx_hbm, table_hbm, out_hbm, idx_v, rows_v, pool_v, sem):
        c = lax.axis_index("c")
        s = lax.axis_index("s")
        w = s * 2 + c
        base = w * (steps * sents_per_step)

        def body(j, carry):
            pltpu.sync_copy(idx_hbm.at[w, j], idx_v)
            pltpu.async_copy(table_hbm.at[idx_v], rows_v, sem).wait()
            for snt in range(sents_per_step):
                for kk in range(d // 16):
                    col = pl.ds(16 * kk, 16)
                    acc = rows_v[seq * snt, col]
                    for r in range(1, seq):
                        acc = acc + rows_v[seq * snt + r, col]
                    pool_v[snt, col] = acc
            pltpu.sync_copy(
                pool_v, out_hbm.at[pl.ds(base + j * sents_per_step, sents_per_step)]
            )
            return carry

        lax.fori_loop(0, steps, body, 0)

    return k(idx3, table)


@functools.partial(jax.jit, static_argnames=("seq",))
def _project_cosine(pooled1, pooled2, Wt, b2, *, seq):
    B, d = pooled1.shape
    p = Wt.shape[1]
    blk = 1024
    inv = 1.0 / float(seq)

    def body(s1_ref, s2_ref, wt_ref, b_ref, out_ref):
        wt = wt_ref[...]
        bb = b_ref[...]
        s1 = s1_ref[...] * inv
        s2 = s2_ref[...] * inv
        p1 = jnp.dot(s1, wt, preferred_element_type=jnp.float32) + bb
        p2 = jnp.dot(s2, wt, preferred_element_type=jnp.float32) + bb
        d12 = jnp.sum(p1 * p2, axis=1)
        n1 = jnp.maximum(jnp.sqrt(jnp.sum(p1 * p1, axis=1)), 1e-8)
        n2 = jnp.maximum(jnp.sqrt(jnp.sum(p2 * p2, axis=1)), 1e-8)
        out_ref[...] = (d12 / (n1 * n2)).reshape(blk, 1)

    out = pl.pallas_call(
        body,
        grid=(B // blk,),
        in_specs=[
            pl.BlockSpec((blk, d), lambda i: (i, 0)),
            pl.BlockSpec((blk, d), lambda i: (i, 0)),
            pl.BlockSpec((d, p), lambda i: (0, 0)),
            pl.BlockSpec((1, p), lambda i: (0, 0)),
        ],
        out_specs=pl.BlockSpec((blk, 1), lambda i: (i, 0)),
        out_shape=jax.ShapeDtypeStruct((B, 1), jnp.float32),
    )(pooled1, pooled2, Wt, b2)
    return out.reshape(B)


def kernel(x, table, W, b):
    B, two, seq = x.shape
    assert two == 2
    d = table.shape[1]
    nc, ns = _sc_info()
    nw = nc * ns

    sents = B * 2
    sents_per_step = 2
    steps = sents // (nw * sents_per_step)
    assert steps * nw * sents_per_step == sents

    idx = x.transpose(1, 0, 2).astype(jnp.int32)
    idx3 = idx.reshape(nw, steps, sents_per_step * seq)

    pooled = _gather_pool(
        idx3, table, nw=nw, steps=steps, sents_per_step=sents_per_step, seq=seq, d=d
    )
    pooled1 = pooled[:B]
    pooled2 = pooled[B:]

    Wt = W.T
    b2 = b.reshape(1, -1)
    return _project_cosine(pooled1, pooled2, Wt, b2, seq=seq)


# speedup vs baseline: 2.8101x; 1.1991x over previous
import functools

import jax
import jax.numpy as jnp
from jax import lax
from jax.experimental import pallas as pl
from jax.experimental.pallas import tpu as pltpu
from jax.experimental.pallas import tpu_sc as plsc


def _sc_info():
    try:
        info = plsc.get_sparse_core_info()
        return info.num_cores, info.num_subcores
    except Exception:
        return 2, 16


@functools.partial(jax.jit, static_argnames=("nw", "steps", "sents_per_step", "seq", "d"))
def _gather_pool(idx3, table, *, nw, steps, sents_per_step, seq, d):
    ipg = sents_per_step * seq
    total_sents = nw * steps * sents_per_step
    mesh = plsc.VectorSubcoreMesh(core_axis_name="c", subcore_axis_name="s")

    nbuf = 2
    nflush = 2
    sents_per_worker = steps * sents_per_step
    steps_per_flush = steps // nflush
    sents_per_flush = sents_per_worker // nflush
    assert steps_per_flush % nbuf == 0

    @functools.partial(
        pl.kernel,
        out_type=jax.ShapeDtypeStruct((total_sents, d), jnp.float32),
        mesh=mesh,
        compiler_params=pltpu.CompilerParams(use_tc_tiling_on_sc=False),
        scratch_types=[
            pltpu.VMEM((steps, ipg), jnp.int32),
            pltpu.VMEM((nbuf, ipg, d), jnp.float32),
            pltpu.VMEM((sents_per_flush, d), jnp.float32),
            pltpu.SemaphoreType.DMA,
        ],
    )
    def k(idx_hbm, table_hbm, out_hbm, idx_v, rows_v, pooled_v, sem):
        c = lax.axis_index("c")
        s = lax.axis_index("s")
        w = s * 2 + c

        pltpu.sync_copy(idx_hbm.at[w], idx_v)
        for b in range(nbuf):
            pltpu.async_copy(table_hbm.at[idx_v.at[b]], rows_v.at[b], sem)

        for h in range(nflush):

            def body(g, carry, h=h):
                for b in range(nbuf):
                    j = h * steps_per_flush + nbuf * g + b
                    jloc = nbuf * g + b
                    pltpu.make_async_copy(
                        table_hbm.at[idx_v.at[j]], rows_v.at[b], sem
                    ).wait()
                    for snt in range(sents_per_step):
                        srow = seq * snt
                        for kk in range(d // 16):
                            col = pl.ds(16 * kk, 16)
                            acc = rows_v[b, srow, col]
                            for r in range(1, seq):
                                acc = acc + rows_v[b, srow + r, col]
                            pooled_v[sents_per_step * jloc + snt, col] = acc

                    @pl.when(j + nbuf < steps)
                    def _():
                        pltpu.async_copy(
                            table_hbm.at[idx_v.at[j + nbuf]], rows_v.at[b], sem
                        )
                return carry

            lax.fori_loop(0, steps_per_flush // nbuf, body, 0)
            pltpu.sync_copy(
                pooled_v,
                out_hbm.at[
                    pl.ds(w * sents_per_worker + h * sents_per_flush, sents_per_flush)
                ],
            )

    return k(idx3, table)


@functools.partial(jax.jit, static_argnames=("seq",))
def _project_cosine(pooled1, pooled2, Wt, b2, *, seq):
    B, d = pooled1.shape
    p = Wt.shape[1]
    blk = 1024
    inv = 1.0 / float(seq)

    def body(s1_ref, s2_ref, wt_ref, b_ref, out_ref):
        wt = wt_ref[...]
        bb = b_ref[...]
        s1 = s1_ref[...] * inv
        s2 = s2_ref[...] * inv
        p1 = jnp.dot(s1, wt, preferred_element_type=jnp.float32) + bb
        p2 = jnp.dot(s2, wt, preferred_element_type=jnp.float32) + bb
        d12 = jnp.sum(p1 * p2, axis=1)
        n1 = jnp.maximum(jnp.sqrt(jnp.sum(p1 * p1, axis=1)), 1e-8)
        n2 = jnp.maximum(jnp.sqrt(jnp.sum(p2 * p2, axis=1)), 1e-8)
        out_ref[...] = (d12 / (n1 * n2)).reshape(blk, 1)

    out = pl.pallas_call(
        body,
        grid=(B // blk,),
        in_specs=[
            pl.BlockSpec((blk, d), lambda i: (i, 0)),
            pl.BlockSpec((blk, d), lambda i: (i, 0)),
            pl.BlockSpec((d, p), lambda i: (0, 0)),
            pl.BlockSpec((1, p), lambda i: (0, 0)),
        ],
        out_specs=pl.BlockSpec((blk, 1), lambda i: (i, 0)),
        out_shape=jax.ShapeDtypeStruct((B, 1), jnp.float32),
    )(pooled1, pooled2, Wt, b2)
    return out.reshape(B)


def kernel(x, table, W, b):
    B, two, seq = x.shape
    assert two == 2
    d = table.shape[1]
    nc, ns = _sc_info()
    nw = nc * ns

    sents = B * 2
    sents_per_step = 2
    steps = sents // (nw * sents_per_step)
    assert steps * nw * sents_per_step == sents

    idx = x.transpose(1, 0, 2).astype(jnp.int32)
    idx3 = idx.reshape(nw, steps, sents_per_step * seq)

    pooled = _gather_pool(
        idx3, table, nw=nw, steps=steps, sents_per_step=sents_per_step, seq=seq, d=d
    )
    pooled1 = pooled[:B]
    pooled2 = pooled[B:]

    Wt = W.T
    b2 = b.reshape(1, -1)
    return _project_cosine(pooled1, pooled2, Wt, b2, seq=seq)
